# Initial kernel scaffold; baseline (speedup 1.0000x reference)
import jax
import jax.numpy as jnp
from jax.experimental import pallas as pl


def kernel(x, edge_index, W1, W2, W3, sim_idx):
    raise NotImplementedError("write your pallas kernel here")


# speedup vs baseline: 12.6531x; 12.6531x over previous
import functools

import jax
import jax.numpy as jnp
from jax import lax
from jax.experimental import pallas as pl
from jax.experimental.pallas import tpu as pltpu
from jax.experimental.pallas import tpu_sc as plsc

_NC = 2
_NS = 16
_NW = _NC * _NS
_CHUNK = 128
_ZR = 128
_DEG_W = 16


def _round_up(v, m):
    return (v + m - 1) // m * m


def _fill2d(ref, n_rows, n_cols, value):
    vec = jnp.full((16,), value, jnp.float32)

    @pl.loop(0, n_rows)
    def _(r):
        for c in range(n_cols // 16):
            ref[r, pl.ds(c * 16, 16)] = vec


def _zero_shared_slice(acc, zbuf, sid, rows_per_sub):
    for r in range(rows_per_sub // _ZR):
        pltpu.sync_copy(zbuf, acc.at[pl.ds(sid * rows_per_sub + r * _ZR, _ZR)])


def _deg_call(dst3, n_pad, nch):
    mesh = plsc.VectorSubcoreMesh(core_axis_name="c", subcore_axis_name="s")
    rows_per_sub = n_pad // _NS

    @functools.partial(
        pl.kernel,
        out_type=jax.ShapeDtypeStruct((_NC, n_pad, _DEG_W), jnp.float32),
        mesh=mesh,
        compiler_params=pltpu.CompilerParams(use_tc_tiling_on_sc=False),
        scratch_types=[
            pltpu.VMEM((nch, _CHUNK), jnp.int32),
            pltpu.VMEM((_CHUNK, _DEG_W), jnp.float32),
            pltpu.VMEM((_ZR, _DEG_W), jnp.float32),
            pltpu.VMEM_SHARED((n_pad, _DEG_W), jnp.float32),
        ],
    )
    def deg_kernel(dst_hbm, out_hbm, dst_v, ones_v, zbuf, acc):
        cid = lax.axis_index("c")
        sid = lax.axis_index("s")
        wid = cid * _NS + sid
        _fill2d(ones_v, _CHUNK, _DEG_W, 1.0)
        _fill2d(zbuf, _ZR, _DEG_W, 0.0)
        _zero_shared_slice(acc, zbuf, sid, rows_per_sub)
        pltpu.sync_copy(dst_hbm.at[wid], dst_v)
        plsc.subcore_barrier()

        @pl.loop(0, nch)
        def _(j):
            pltpu.sync_copy(ones_v, acc.at[dst_v.at[j]], add=True)

        plsc.subcore_barrier()
        sl = pl.ds(sid * rows_per_sub, rows_per_sub)
        pltpu.sync_copy(acc.at[sl], out_hbm.at[cid, sl])

    return deg_kernel(dst3)


def _scatter_rows_call(g, src3, dst3, n_pad, nch, h):
    mesh = plsc.VectorSubcoreMesh(core_axis_name="c", subcore_axis_name="s")
    rows_per_sub = n_pad // _NS

    @functools.partial(
        pl.kernel,
        out_type=jax.ShapeDtypeStruct((_NC, n_pad, h), jnp.float32),
        mesh=mesh,
        compiler_params=pltpu.CompilerParams(use_tc_tiling_on_sc=False),
        scratch_types=[
            pltpu.VMEM((nch, _CHUNK), jnp.int32),
            pltpu.VMEM((nch, _CHUNK), jnp.int32),
            pltpu.VMEM((2, _CHUNK, h), jnp.float32),
            pltpu.VMEM((_ZR, h), jnp.float32),
            pltpu.SemaphoreType.DMA((2,)),
            pltpu.VMEM_SHARED((n_pad, h), jnp.float32),
        ],
    )
    def scat_kernel(g_hbm, src_hbm, dst_hbm, out_hbm, src_v, dst_v, rows_v,
                    zbuf, sems, acc):
        cid = lax.axis_index("c")
        sid = lax.axis_index("s")
        wid = cid * _NS + sid
        _fill2d(zbuf, _ZR, h, 0.0)
        _zero_shared_slice(acc, zbuf, sid, rows_per_sub)
        pltpu.sync_copy(src_hbm.at[wid], src_v)
        pltpu.sync_copy(dst_hbm.at[wid], dst_v)
        plsc.subcore_barrier()

        for b in range(2):
            pltpu.async_copy(g_hbm.at[src_v.at[b]], rows_v.at[b], sems.at[b])

        @pl.loop(0, nch, step=2)
        def _(i):
            for b in range(2):
                j = i + b
                pltpu.make_async_copy(
                    g_hbm.at[src_v.at[j]], rows_v.at[b], sems.at[b]).wait()
                pltpu.sync_copy(rows_v.at[b], acc.at[dst_v.at[j]], add=True)

                @pl.when(j + 2 < nch)
                def _():
                    pltpu.async_copy(
                        g_hbm.at[src_v.at[j + 2]], rows_v.at[b], sems.at[b])

        plsc.subcore_barrier()
        sl = pl.ds(sid * rows_per_sub, rows_per_sub)
        pltpu.sync_copy(acc.at[sl], out_hbm.at[cid, sl])

    return scat_kernel(g, src3, dst3)


def _dinv_of(degs_blk):
    deg = degs_blk[0, :, 0:1] + degs_blk[1, :, 0:1] + 1.0
    return lax.rsqrt(jnp.maximum(deg, 1.0))


def _mm_scale_kernel(x_ref, w_ref, degs_ref, o_ref):
    dinv = _dinv_of(degs_ref)
    o_ref[...] = jnp.dot(x_ref[...], w_ref[...],
                         preferred_element_type=jnp.float32) * dinv


def _mid_kernel(parts_a_ref, parts_b_ref, g_ref, degs_ref, w_ref, o_ref):
    dinv = _dinv_of(degs_ref)
    ps = jnp.concatenate(
        [parts_a_ref[0] + parts_a_ref[1], parts_b_ref[0] + parts_b_ref[1]],
        axis=1)
    pre = (ps + g_ref[...]) * dinv
    h = jnp.maximum(pre, 0.0)
    o_ref[...] = jnp.dot(h, w_ref[...],
                         preferred_element_type=jnp.float32) * dinv


def _fin_kernel(parts_ref, g_ref, degs_ref, w_ref, z_ref, zt_ref):
    dinv = _dinv_of(degs_ref)
    pre = (parts_ref[0] + parts_ref[1] + g_ref[...]) * dinv
    h = jnp.maximum(pre, 0.0)
    z = jnp.dot(h, w_ref[...], preferred_element_type=jnp.float32)
    z_ref[...] = z
    zt_ref[...] = z.T


def _zzt_kernel(z_ref, zt_ref, o_ref):
    o_ref[...] = jnp.dot(z_ref[...], zt_ref[...],
                         preferred_element_type=jnp.float32)


def kernel(x, edge_index, W1, W2, W3, sim_idx):
    n, d_in = x.shape
    h1 = W1.shape[1]
    h2 = W2.shape[1]
    e = edge_index.shape[1]

    n_pad = _round_up(n, 2048)
    if n_pad == n:
        n_pad += 2048
    e_pad = _round_up(e, _NW * _CHUNK * 2)
    nch = e_pad // (_NW * _CHUNK)

    src = edge_index[0]
    dst = edge_index[1]
    pad = e_pad - e
    pidx = jnp.arange(pad, dtype=jnp.int32)
    srcp = jnp.concatenate([src, pidx % n])
    dstp = jnp.concatenate([dst, n + pidx % (n_pad - n)])
    src3 = srcp.reshape(_NW, nch, _CHUNK)
    dst3 = dstp.reshape(_NW, nch, _CHUNK)

    degs = _deg_call(dst3, n_pad, nch)

    bm = 1024
    grid_m = pl.cdiv(n, bm)

    g1 = pl.pallas_call(
        _mm_scale_kernel,
        grid=(grid_m,),
        in_specs=[
            pl.BlockSpec((bm, d_in), lambda i: (i, 0)),
            pl.BlockSpec((d_in, h1), lambda i: (0, 0)),
            pl.BlockSpec((2, bm, _DEG_W), lambda i: (0, i, 0)),
        ],
        out_specs=pl.BlockSpec((bm, h1), lambda i: (i, 0)),
        out_shape=jax.ShapeDtypeStruct((n, h1), jnp.float32),
    )(x, W1, degs)

    hh = h1 // 2
    parts1a = _scatter_rows_call(g1[:, :hh], src3, dst3, n_pad, nch, hh)
    parts1b = _scatter_rows_call(g1[:, hh:], src3, dst3, n_pad, nch, hh)

    g2 = pl.pallas_call(
        _mid_kernel,
        grid=(grid_m,),
        in_specs=[
            pl.BlockSpec((2, bm, hh), lambda i: (0, i, 0)),
            pl.BlockSpec((2, bm, hh), lambda i: (0, i, 0)),
            pl.BlockSpec((bm, h1), lambda i: (i, 0)),
            pl.BlockSpec((2, bm, _DEG_W), lambda i: (0, i, 0)),
            pl.BlockSpec((h1, h2), lambda i: (0, 0)),
        ],
        out_specs=pl.BlockSpec((bm, h2), lambda i: (i, 0)),
        out_shape=jax.ShapeDtypeStruct((n, h2), jnp.float32),
    )(parts1a, parts1b, g1, degs, W2)

    parts2 = _scatter_rows_call(g2, src3, dst3, n_pad, nch, h2)

    w3s = W3[sim_idx]
    z, zt = pl.pallas_call(
        _fin_kernel,
        grid=(grid_m,),
        in_specs=[
            pl.BlockSpec((2, bm, h2), lambda i: (0, i, 0)),
            pl.BlockSpec((bm, h2), lambda i: (i, 0)),
            pl.BlockSpec((2, bm, _DEG_W), lambda i: (0, i, 0)),
            pl.BlockSpec((h2, h2), lambda i: (0, 0)),
        ],
        out_specs=[
            pl.BlockSpec((bm, h2), lambda i: (i, 0)),
            pl.BlockSpec((h2, bm), lambda i: (0, i)),
        ],
        out_shape=[
            jax.ShapeDtypeStruct((n, h2), jnp.float32),
            jax.ShapeDtypeStruct((h2, n), jnp.float32),
        ],
    )(parts2, g2, degs, w3s)

    recon = pl.pallas_call(
        _zzt_kernel,
        grid=(grid_m, grid_m),
        in_specs=[
            pl.BlockSpec((bm, h2), lambda i, j: (i, 0)),
            pl.BlockSpec((h2, bm), lambda i, j: (0, j)),
        ],
        out_specs=pl.BlockSpec((bm, bm), lambda i, j: (i, j)),
        out_shape=jax.ShapeDtypeStruct((n, n), jnp.float32),
    )(z, zt)

    return recon.reshape(-1)


# speedup vs baseline: 12.6554x; 1.0002x over previous
import functools

import jax
import jax.numpy as jnp
from jax import lax
from jax.experimental import pallas as pl
from jax.experimental.pallas import tpu as pltpu
from jax.experimental.pallas import tpu_sc as plsc

_NC = 2
_NS = 16
_CHUNK = 128
_ZR = 128
_DEG_W = 16

_SC_PARAMS = pltpu.CompilerParams(use_tc_tiling_on_sc=False)


def _round_up(v, m):
    return (v + m - 1) // m * m


def _fill2d(ref, n_rows, n_cols, value):
    vec = jnp.full((16,), value, jnp.float32)

    @pl.loop(0, n_rows)
    def _(r):
        for c in range(n_cols // 16):
            ref[r, pl.ds(c * 16, 16)] = vec


def _zero_shared_slice(acc, zbuf, sid, rows_per_sub):
    for r in range(rows_per_sub // _ZR):
        pltpu.sync_copy(zbuf, acc.at[pl.ds(sid * rows_per_sub + r * _ZR, _ZR)])


def _deg_call(dst3, n_pad, nch):
    mesh = plsc.VectorSubcoreMesh(core_axis_name="c", subcore_axis_name="s")
    rows_per_sub = n_pad // _NS
    half = nch // 2

    @functools.partial(
        pl.kernel,
        out_type=jax.ShapeDtypeStruct((_NC, n_pad, _DEG_W), jnp.float32),
        mesh=mesh,
        compiler_params=_SC_PARAMS,
        scratch_types=[
            pltpu.VMEM((half, _CHUNK), jnp.int32),
            pltpu.VMEM((_CHUNK, _DEG_W), jnp.float32),
            pltpu.VMEM((_ZR, _DEG_W), jnp.float32),
            pltpu.VMEM_SHARED((n_pad, _DEG_W), jnp.float32),
        ],
    )
    def deg_kernel(dst_hbm, out_hbm, dst_v, ones_v, zbuf, acc):
        cid = lax.axis_index("c")
        sid = lax.axis_index("s")
        _fill2d(ones_v, _CHUNK, _DEG_W, 1.0)
        _fill2d(zbuf, _ZR, _DEG_W, 0.0)
        _zero_shared_slice(acc, zbuf, sid, rows_per_sub)
        pltpu.sync_copy(dst_hbm.at[sid, pl.ds(cid * half, half)], dst_v)
        plsc.subcore_barrier()

        @pl.loop(0, half)
        def _(j):
            pltpu.sync_copy(ones_v, acc.at[dst_v.at[j]], add=True)

        plsc.subcore_barrier()
        sl = pl.ds(sid * rows_per_sub, rows_per_sub)
        pltpu.sync_copy(acc.at[sl], out_hbm.at[cid, sl])

    return deg_kernel(dst3)


def _scatter_cols_split_call(ga, gb, src3, dst3, n_pad, nch, hh):
    mesh = plsc.VectorSubcoreMesh(core_axis_name="c", subcore_axis_name="s")
    rows_per_sub = n_pad // _NS

    @functools.partial(
        pl.kernel,
        out_type=jax.ShapeDtypeStruct((_NC, n_pad, hh), jnp.float32),
        mesh=mesh,
        compiler_params=_SC_PARAMS,
        scratch_types=[
            pltpu.VMEM((nch, _CHUNK), jnp.int32),
            pltpu.VMEM((nch, _CHUNK), jnp.int32),
            pltpu.VMEM((2, _CHUNK, hh), jnp.float32),
            pltpu.VMEM((_ZR, hh), jnp.float32),
            pltpu.SemaphoreType.DMA((2,)),
            pltpu.VMEM_SHARED((n_pad, hh), jnp.float32),
        ],
    )
    def scat_kernel(ga_hbm, gb_hbm, src_hbm, dst_hbm, out_hbm, src_v, dst_v,
                    rows_v, zbuf, sems, acc):
        cid = lax.axis_index("c")
        sid = lax.axis_index("s")
        _fill2d(zbuf, _ZR, hh, 0.0)
        _zero_shared_slice(acc, zbuf, sid, rows_per_sub)
        pltpu.sync_copy(src_hbm.at[sid], src_v)
        pltpu.sync_copy(dst_hbm.at[sid], dst_v)
        plsc.subcore_barrier()

        def run(g_hbm):
            for b in range(2):
                pltpu.async_copy(g_hbm.at[src_v.at[b]], rows_v.at[b],
                                 sems.at[b])

            @pl.loop(0, nch, step=2)
            def _(i):
                for b in range(2):
                    j = i + b
                    pltpu.make_async_copy(g_hbm.at[src_v.at[j]], rows_v.at[b],
                                          sems.at[b]).wait()
                    pltpu.sync_copy(rows_v.at[b], acc.at[dst_v.at[j]],
                                    add=True)

                    @pl.when(j + 2 < nch)
                    def _():
                        pltpu.async_copy(g_hbm.at[src_v.at[j + 2]],
                                         rows_v.at[b], sems.at[b])

        @pl.when(cid == 0)
        def _():
            run(ga_hbm)

        @pl.when(cid == 1)
        def _():
            run(gb_hbm)

        plsc.subcore_barrier()
        sl = pl.ds(sid * rows_per_sub, rows_per_sub)
        pltpu.sync_copy(acc.at[sl], out_hbm.at[cid, sl])

    return scat_kernel(ga, gb, src3, dst3)


def _dinv_of(degs_blk):
    deg = degs_blk[0, :, 0:1] + degs_blk[1, :, 0:1] + 1.0
    return lax.rsqrt(jnp.maximum(deg, 1.0))


def _mm_scale_kernel(x_ref, w_ref, degs_ref, o_ref):
    dinv = _dinv_of(degs_ref)
    o_ref[...] = jnp.dot(x_ref[...], w_ref[...],
                         preferred_element_type=jnp.float32) * dinv


def _mid_kernel(parts_ref, g_ref, degs_ref, w_ref, o_ref):
    dinv = _dinv_of(degs_ref)
    ps = jnp.concatenate([parts_ref[0], parts_ref[1]], axis=1)
    pre = (ps + g_ref[...]) * dinv
    h = jnp.maximum(pre, 0.0)
    o_ref[...] = jnp.dot(h, w_ref[...],
                         preferred_element_type=jnp.float32) * dinv


def _fin_kernel(parts_ref, g_ref, degs_ref, w_ref, z_ref, zt_ref):
    dinv = _dinv_of(degs_ref)
    ps = jnp.concatenate([parts_ref[0], parts_ref[1]], axis=1)
    pre = (ps + g_ref[...]) * dinv
    h = jnp.maximum(pre, 0.0)
    z = jnp.dot(h, w_ref[...], preferred_element_type=jnp.float32)
    z_ref[...] = z
    zt_ref[...] = z.T


def _zzt_kernel(z_ref, zt_ref, o_ref):
    o_ref[...] = jnp.dot(z_ref[...], zt_ref[...],
                         preferred_element_type=jnp.float32)


def kernel(x, edge_index, W1, W2, W3, sim_idx):
    n, d_in = x.shape
    h1 = W1.shape[1]
    h2 = W2.shape[1]
    e = edge_index.shape[1]

    n_pad = _round_up(n, 2048)
    if n_pad == n:
        n_pad += 2048
    e_pad = _round_up(e, _NS * _CHUNK * 4)
    nch = e_pad // (_NS * _CHUNK)

    src = edge_index[0]
    dst = edge_index[1]
    pad = e_pad - e
    pidx = jnp.arange(pad, dtype=jnp.int32)
    srcp = jnp.concatenate([src, pidx % n])
    dstp = jnp.concatenate([dst, n + pidx % (n_pad - n)])
    src3 = srcp.reshape(_NS, nch, _CHUNK)
    dst3 = dstp.reshape(_NS, nch, _CHUNK)

    degs = _deg_call(dst3, n_pad, nch)

    bm = 1024
    grid_m = pl.cdiv(n, bm)

    g1 = pl.pallas_call(
        _mm_scale_kernel,
        grid=(grid_m,),
        in_specs=[
            pl.BlockSpec((bm, d_in), lambda i: (i, 0)),
            pl.BlockSpec((d_in, h1), lambda i: (0, 0)),
            pl.BlockSpec((2, bm, _DEG_W), lambda i: (0, i, 0)),
        ],
        out_specs=pl.BlockSpec((bm, h1), lambda i: (i, 0)),
        out_shape=jax.ShapeDtypeStruct((n, h1), jnp.float32),
    )(x, W1, degs)

    hh1 = h1 // 2
    parts1 = _scatter_cols_split_call(
        g1[:, :hh1], g1[:, hh1:], src3, dst3, n_pad, nch, hh1)

    g2 = pl.pallas_call(
        _mid_kernel,
        grid=(grid_m,),
        in_specs=[
            pl.BlockSpec((2, bm, hh1), lambda i: (0, i, 0)),
            pl.BlockSpec((bm, h1), lambda i: (i, 0)),
            pl.BlockSpec((2, bm, _DEG_W), lambda i: (0, i, 0)),
            pl.BlockSpec((h1, h2), lambda i: (0, 0)),
        ],
        out_specs=pl.BlockSpec((bm, h2), lambda i: (i, 0)),
        out_shape=jax.ShapeDtypeStruct((n, h2), jnp.float32),
    )(parts1, g1, degs, W2)

    hh2 = h2 // 2
    parts2 = _scatter_cols_split_call(
        g2[:, :hh2], g2[:, hh2:], src3, dst3, n_pad, nch, hh2)

    w3s = W3[sim_idx]
    z, zt = pl.pallas_call(
        _fin_kernel,
        grid=(grid_m,),
        in_specs=[
            pl.BlockSpec((2, bm, hh2), lambda i: (0, i, 0)),
            pl.BlockSpec((bm, h2), lambda i: (i, 0)),
            pl.BlockSpec((2, bm, _DEG_W), lambda i: (0, i, 0)),
            pl.BlockSpec((h2, h2), lambda i: (0, 0)),
        ],
        out_specs=[
            pl.BlockSpec((bm, h2), lambda i: (i, 0)),
            pl.BlockSpec((h2, bm), lambda i: (0, i)),
        ],
        out_shape=[
            jax.ShapeDtypeStruct((n, h2), jnp.float32),
            jax.ShapeDtypeStruct((h2, n), jnp.float32),
        ],
    )(parts2, g2, degs, w3s)

    bmz = 256
    recon = pl.pallas_call(
        _zzt_kernel,
        grid=(pl.cdiv(n, bmz),),
        in_specs=[
            pl.BlockSpec((bmz, h2), lambda i: (i, 0)),
            pl.BlockSpec((h2, n), lambda i: (0, 0)),
        ],
        out_specs=pl.BlockSpec((bmz, n), lambda i: (i, 0)),
        out_shape=jax.ShapeDtypeStruct((n, n), jnp.float32),
    )(z, zt)

    return recon.reshape(-1)


# speedup vs baseline: 13.7603x; 1.0873x over previous
import functools

import jax
import jax.numpy as jnp
from jax import lax
from jax.experimental import pallas as pl
from jax.experimental.pallas import tpu as pltpu
from jax.experimental.pallas import tpu_sc as plsc

_NC = 2
_NS = 16
_CHUNK = 128
_ZR = 128
_DEG_W = 16

_SC_PARAMS = pltpu.CompilerParams(use_tc_tiling_on_sc=False)


def _round_up(v, m):
    return (v + m - 1) // m * m


def _fill2d(ref, n_rows, n_cols, value):
    vec = jnp.full((16,), value, jnp.float32)

    @pl.loop(0, n_rows)
    def _(r):
        for c in range(n_cols // 16):
            ref[r, pl.ds(c * 16, 16)] = vec


def _zero_shared_slice(acc, zbuf, sid, rows_per_sub):
    for r in range(rows_per_sub // _ZR):
        pltpu.sync_copy(zbuf, acc.at[pl.ds(sid * rows_per_sub + r * _ZR, _ZR)])


_NBUF = 4


def _gather_scatter_pipe(g_hbm, acc, src_v, dst_v, rows_v, gsems, ssems,
                         count):
    for b in range(_NBUF):
        pltpu.async_copy(g_hbm.at[src_v.at[b]], rows_v.at[b], gsems.at[b])

    @pl.loop(0, count, step=_NBUF)
    def _(i):
        for b in range(_NBUF):
            j = i + b
            pltpu.make_async_copy(g_hbm.at[src_v.at[j]], rows_v.at[b],
                                  gsems.at[b]).wait()
            pltpu.async_copy(rows_v.at[b], acc.at[dst_v.at[j]], ssems.at[b],
                             add=True)
        for b in range(_NBUF):
            j = i + b

            @pl.when(j + _NBUF < count)
            def _():
                pltpu.make_async_copy(rows_v.at[b], acc.at[dst_v.at[j]],
                                      ssems.at[b]).wait()
                pltpu.async_copy(g_hbm.at[src_v.at[j + _NBUF]], rows_v.at[b],
                                 gsems.at[b])

    for b in range(_NBUF):
        pltpu.make_async_copy(rows_v.at[b], acc.at[dst_v.at[0]],
                              ssems.at[b]).wait()


def _deg_call(dst3, n_pad, nch):
    mesh = plsc.VectorSubcoreMesh(core_axis_name="c", subcore_axis_name="s")
    rows_per_sub = n_pad // _NS
    half = nch // 2

    @functools.partial(
        pl.kernel,
        out_type=jax.ShapeDtypeStruct((_NC, n_pad, _DEG_W), jnp.float32),
        mesh=mesh,
        compiler_params=_SC_PARAMS,
        scratch_types=[
            pltpu.VMEM((half, _CHUNK), jnp.int32),
            pltpu.VMEM((_CHUNK, _DEG_W), jnp.float32),
            pltpu.VMEM((_ZR, _DEG_W), jnp.float32),
            pltpu.VMEM_SHARED((n_pad, _DEG_W), jnp.float32),
        ],
    )
    def deg_kernel(dst_hbm, out_hbm, dst_v, ones_v, zbuf, acc):
        cid = lax.axis_index("c")
        sid = lax.axis_index("s")
        _fill2d(ones_v, _CHUNK, _DEG_W, 1.0)
        _fill2d(zbuf, _ZR, _DEG_W, 0.0)
        _zero_shared_slice(acc, zbuf, sid, rows_per_sub)
        pltpu.sync_copy(dst_hbm.at[sid, pl.ds(cid * half, half)], dst_v)
        plsc.subcore_barrier()

        @pl.loop(0, half)
        def _(j):
            pltpu.sync_copy(ones_v, acc.at[dst_v.at[j]], add=True)

        plsc.subcore_barrier()
        sl = pl.ds(sid * rows_per_sub, rows_per_sub)
        pltpu.sync_copy(acc.at[sl], out_hbm.at[cid, sl])

    return deg_kernel(dst3)


def _scatter_cols_split_call(ga, gb, src3, dst3, n_pad, nch, hh):
    mesh = plsc.VectorSubcoreMesh(core_axis_name="c", subcore_axis_name="s")
    rows_per_sub = n_pad // _NS

    @functools.partial(
        pl.kernel,
        out_type=jax.ShapeDtypeStruct((_NC, n_pad, hh), jnp.float32),
        mesh=mesh,
        compiler_params=_SC_PARAMS,
        scratch_types=[
            pltpu.VMEM((nch, _CHUNK), jnp.int32),
            pltpu.VMEM((nch, _CHUNK), jnp.int32),
            pltpu.VMEM((_NBUF, _CHUNK, hh), jnp.float32),
            pltpu.VMEM((_ZR, hh), jnp.float32),
            pltpu.SemaphoreType.DMA((_NBUF,)),
            pltpu.SemaphoreType.DMA((_NBUF,)),
            pltpu.VMEM_SHARED((n_pad, hh), jnp.float32),
        ],
    )
    def scat_kernel(ga_hbm, gb_hbm, src_hbm, dst_hbm, out_hbm, src_v, dst_v,
                    rows_v, zbuf, gsems, ssems, acc):
        cid = lax.axis_index("c")
        sid = lax.axis_index("s")
        _fill2d(zbuf, _ZR, hh, 0.0)
        _zero_shared_slice(acc, zbuf, sid, rows_per_sub)
        pltpu.sync_copy(src_hbm.at[sid], src_v)
        pltpu.sync_copy(dst_hbm.at[sid], dst_v)
        plsc.subcore_barrier()

        def run(g_hbm):
            _gather_scatter_pipe(g_hbm, acc, src_v, dst_v, rows_v, gsems,
                                 ssems, nch)

        @pl.when(cid == 0)
        def _():
            run(ga_hbm)

        @pl.when(cid == 1)
        def _():
            run(gb_hbm)

        plsc.subcore_barrier()
        sl = pl.ds(sid * rows_per_sub, rows_per_sub)
        pltpu.sync_copy(acc.at[sl], out_hbm.at[cid, sl])

    return scat_kernel(ga, gb, src3, dst3)


def _scatter_rows_edge_split_call(g, src3, dst3, n_pad, nch, h):
    mesh = plsc.VectorSubcoreMesh(core_axis_name="c", subcore_axis_name="s")
    rows_per_sub = n_pad // _NS
    half = nch // 2

    @functools.partial(
        pl.kernel,
        out_type=jax.ShapeDtypeStruct((_NC, n_pad, h), jnp.float32),
        mesh=mesh,
        compiler_params=_SC_PARAMS,
        scratch_types=[
            pltpu.VMEM((half, _CHUNK), jnp.int32),
            pltpu.VMEM((half, _CHUNK), jnp.int32),
            pltpu.VMEM((_NBUF, _CHUNK, h), jnp.float32),
            pltpu.VMEM((_ZR, h), jnp.float32),
            pltpu.SemaphoreType.DMA((_NBUF,)),
            pltpu.SemaphoreType.DMA((_NBUF,)),
            pltpu.VMEM_SHARED((n_pad, h), jnp.float32),
        ],
    )
    def scat_kernel(g_hbm, src_hbm, dst_hbm, out_hbm, src_v, dst_v, rows_v,
                    zbuf, gsems, ssems, acc):
        cid = lax.axis_index("c")
        sid = lax.axis_index("s")
        _fill2d(zbuf, _ZR, h, 0.0)
        _zero_shared_slice(acc, zbuf, sid, rows_per_sub)
        pltpu.sync_copy(src_hbm.at[sid, pl.ds(cid * half, half)], src_v)
        pltpu.sync_copy(dst_hbm.at[sid, pl.ds(cid * half, half)], dst_v)
        plsc.subcore_barrier()
        _gather_scatter_pipe(g_hbm, acc, src_v, dst_v, rows_v, gsems, ssems,
                             half)
        plsc.subcore_barrier()
        sl = pl.ds(sid * rows_per_sub, rows_per_sub)
        pltpu.sync_copy(acc.at[sl], out_hbm.at[cid, sl])

    return scat_kernel(g, src3, dst3)


def _dinv_of(degs_blk):
    deg = degs_blk[0, :, 0:1] + degs_blk[1, :, 0:1] + 1.0
    return lax.rsqrt(jnp.maximum(deg, 1.0))


def _mm_scale_kernel(x_ref, w_ref, degs_ref, o_ref):
    dinv = _dinv_of(degs_ref)
    o_ref[...] = jnp.dot(x_ref[...], w_ref[...],
                         preferred_element_type=jnp.float32) * dinv


def _mid_kernel(parts_ref, g_ref, degs_ref, w_ref, o_ref):
    dinv = _dinv_of(degs_ref)
    ps = jnp.concatenate([parts_ref[0], parts_ref[1]], axis=1)
    pre = (ps + g_ref[...]) * dinv
    h = jnp.maximum(pre, 0.0)
    o_ref[...] = jnp.dot(h, w_ref[...],
                         preferred_element_type=jnp.float32) * dinv


def _fin_kernel(parts_ref, g_ref, degs_ref, w_ref, z_ref, zt_ref):
    dinv = _dinv_of(degs_ref)
    ps = parts_ref[0] + parts_ref[1]
    pre = (ps + g_ref[...]) * dinv
    h = jnp.maximum(pre, 0.0)
    z = jnp.dot(h, w_ref[...], preferred_element_type=jnp.float32)
    z_ref[...] = z
    zt_ref[...] = z.T


def _zzt_kernel(z_ref, zt_ref, o_ref):
    o_ref[...] = jnp.dot(z_ref[...], zt_ref[...],
                         preferred_element_type=jnp.float32)


def kernel(x, edge_index, W1, W2, W3, sim_idx):
    n, d_in = x.shape
    h1 = W1.shape[1]
    h2 = W2.shape[1]
    e = edge_index.shape[1]

    n_pad = _round_up(n, 2048)
    if n_pad == n:
        n_pad += 2048
    e_pad = _round_up(e, _NS * _CHUNK * 4)
    nch = e_pad // (_NS * _CHUNK)

    src = edge_index[0]
    dst = edge_index[1]
    pad = e_pad - e
    pidx = jnp.arange(pad, dtype=jnp.int32)
    srcp = jnp.concatenate([src, pidx % n])
    dstp = jnp.concatenate([dst, n + pidx % (n_pad - n)])
    src3 = srcp.reshape(_NS, nch, _CHUNK)
    dst3 = dstp.reshape(_NS, nch, _CHUNK)

    degs = _deg_call(dst3, n_pad, nch)

    bm = 1024
    grid_m = pl.cdiv(n, bm)

    g1 = pl.pallas_call(
        _mm_scale_kernel,
        grid=(grid_m,),
        in_specs=[
            pl.BlockSpec((bm, d_in), lambda i: (i, 0)),
            pl.BlockSpec((d_in, h1), lambda i: (0, 0)),
            pl.BlockSpec((2, bm, _DEG_W), lambda i: (0, i, 0)),
        ],
        out_specs=pl.BlockSpec((bm, h1), lambda i: (i, 0)),
        out_shape=jax.ShapeDtypeStruct((n, h1), jnp.float32),
    )(x, W1, degs)

    hh1 = h1 // 2
    parts1 = _scatter_cols_split_call(
        g1[:, :hh1], g1[:, hh1:], src3, dst3, n_pad, nch, hh1)

    g2 = pl.pallas_call(
        _mid_kernel,
        grid=(grid_m,),
        in_specs=[
            pl.BlockSpec((2, bm, hh1), lambda i: (0, i, 0)),
            pl.BlockSpec((bm, h1), lambda i: (i, 0)),
            pl.BlockSpec((2, bm, _DEG_W), lambda i: (0, i, 0)),
            pl.BlockSpec((h1, h2), lambda i: (0, 0)),
        ],
        out_specs=pl.BlockSpec((bm, h2), lambda i: (i, 0)),
        out_shape=jax.ShapeDtypeStruct((n, h2), jnp.float32),
    )(parts1, g1, degs, W2)

    parts2 = _scatter_rows_edge_split_call(g2, src3, dst3, n_pad, nch, h2)

    w3s = W3[sim_idx]
    z, zt = pl.pallas_call(
        _fin_kernel,
        grid=(grid_m,),
        in_specs=[
            pl.BlockSpec((2, bm, h2), lambda i: (0, i, 0)),
            pl.BlockSpec((bm, h2), lambda i: (i, 0)),
            pl.BlockSpec((2, bm, _DEG_W), lambda i: (0, i, 0)),
            pl.BlockSpec((h2, h2), lambda i: (0, 0)),
        ],
        out_specs=[
            pl.BlockSpec((bm, h2), lambda i: (i, 0)),
            pl.BlockSpec((h2, bm), lambda i: (0, i)),
        ],
        out_shape=[
            jax.ShapeDtypeStruct((n, h2), jnp.float32),
            jax.ShapeDtypeStruct((h2, n), jnp.float32),
        ],
    )(parts2, g2, degs, w3s)

    bmz = 256
    recon = pl.pallas_call(
        _zzt_kernel,
        grid=(pl.cdiv(n, bmz),),
        in_specs=[
            pl.BlockSpec((bmz, h2), lambda i: (i, 0)),
            pl.BlockSpec((h2, n), lambda i: (0, 0)),
        ],
        out_specs=pl.BlockSpec((bmz, n), lambda i: (i, 0)),
        out_shape=jax.ShapeDtypeStruct((n, n), jnp.float32),
    )(z, zt)

    return recon.reshape(-1)
